# baseline (speedup 1.0000x reference)
import jax
import jax.numpy as jnp
from jax import lax
from jax.experimental import pallas as pl
from jax.experimental.pallas import tpu as pltpu

NSTREAMS = 2
BLOCKP = 256


def _gate_kernel(*refs):
    x_refs = refs[:NSTREAMS]
    w_ref, b_ref, probs_ref, ent_ref, w_scr, bias_scr, mask_scr, acc_ref = (
        refs[NSTREAMS:])
    i = pl.program_id(0)
    nb = pl.num_programs(0)
    n_exp, in_dim = w_ref.shape

    @pl.when(i == 0)
    def _init():
        wb = w_ref[...].astype(jnp.bfloat16)
        zero = jnp.zeros((n_exp, in_dim), jnp.bfloat16)
        w_scr[:n_exp, :in_dim] = wb
        w_scr[:n_exp, in_dim:] = zero
        w_scr[n_exp:, :in_dim] = zero
        w_scr[n_exp:, in_dim:] = wb
        bias_scr[:, :n_exp] = b_ref[...][None, :]
        bias_scr[:, n_exp:] = b_ref[...][None, :]
        row = lax.broadcasted_iota(jnp.int32, (2 * n_exp, 2 * n_exp), 0)
        col = lax.broadcasted_iota(jnp.int32, (2 * n_exp, 2 * n_exp), 1)
        mask_scr[...] = jnp.where((row // n_exp) == (col // n_exp), 1.0, 0.0)
        acc_ref[0] = 0.0

    w2 = w_scr[...]
    bias = bias_scr[...]
    mask = mask_scr[...]
    block = x_refs[0].shape[0]
    total = jnp.zeros((), jnp.float32)
    for k, x_ref in enumerate(x_refs):
        logits = lax.dot_general(
            x_ref[...].astype(jnp.bfloat16), w2, (((1,), (1,)), ((), ())),
            preferred_element_type=jnp.float32) + bias
        m = jnp.max(logits, axis=-1, keepdims=True)
        e = jnp.exp(logits - m)
        s = jnp.dot(e, mask, preferred_element_type=jnp.float32)
        p = e / s
        probs_ref[k * block:(k + 1) * block, :] = p
        total += jnp.sum(p * jnp.log(p + 1e-9))
    acc_ref[0] += total

    @pl.when(i == nb - 1)
    def _finalize():
        ent_ref[0] = -acc_ref[0] / (nb * NSTREAMS * block * 2)


@jax.jit
def _switch_gate(x, W, b):
    tokens, in_dim = x.shape
    num_experts = W.shape[0]
    prows = tokens // 2
    x2 = x.reshape(prows, 2 * in_dim)
    step_rows = NSTREAMS * BLOCKP
    nb = prows // step_rows

    def _xspec(k):
        return pl.BlockSpec((BLOCKP, 2 * in_dim),
                            lambda i, k=k: (NSTREAMS * i + k, 0))

    probs, ent = pl.pallas_call(
        _gate_kernel,
        grid=(nb,),
        in_specs=[_xspec(k) for k in range(NSTREAMS)] + [
            pl.BlockSpec((num_experts, in_dim), lambda i: (0, 0)),
            pl.BlockSpec((num_experts,), lambda i: (0,)),
        ],
        out_specs=[
            pl.BlockSpec((step_rows, 2 * num_experts), lambda i: (i, 0)),
            pl.BlockSpec(memory_space=pltpu.SMEM),
        ],
        out_shape=[
            jax.ShapeDtypeStruct((prows, 2 * num_experts), jnp.float32),
            jax.ShapeDtypeStruct((1,), jnp.float32),
        ],
        scratch_shapes=[
            pltpu.VMEM((2 * num_experts, 2 * in_dim), jnp.bfloat16),
            pltpu.VMEM((1, 2 * num_experts), jnp.float32),
            pltpu.VMEM((2 * num_experts, 2 * num_experts), jnp.float32),
            pltpu.SMEM((1,), jnp.float32),
        ],
        compiler_params=pltpu.CompilerParams(
            dimension_semantics=("arbitrary",),
        ),
    )(*([x2] * NSTREAMS), W, b)
    return probs.reshape(tokens, num_experts), ent[0]


def kernel(x, W, b):
    return _switch_gate(x, W, b)


# speedup vs baseline: 4.0521x; 4.0521x over previous
import jax
import jax.numpy as jnp
from jax import lax
from jax.experimental import pallas as pl
from jax.experimental.pallas import tpu as pltpu

NSTREAMS = 2
BLOCK = 512


def _softmax_rows(logits):
    m = jnp.max(logits, axis=-1, keepdims=True)
    e = jnp.exp(logits - m)
    s = jnp.sum(e, axis=-1, keepdims=True)
    return e / s


def _gate_kernel(*refs):
    x_refs = refs[:NSTREAMS]
    w_ref, b_ref, probs_ref, ent_ref, w_scr, acc_ref = refs[NSTREAMS:]
    i = pl.program_id(0)
    nb = pl.num_programs(0)
    block = x_refs[0].shape[0]

    @pl.when(i == 0)
    def _init():
        w_scr[...] = w_ref[...].astype(jnp.bfloat16)
        acc_ref[0] = 0.0

    w = w_scr[...]
    bias = b_ref[...][None, :]
    total = jnp.zeros((), jnp.float32)
    for k, x_ref in enumerate(x_refs):
        p = _softmax_rows(lax.dot_general(
            x_ref[...].astype(jnp.bfloat16), w, (((1,), (1,)), ((), ())),
            preferred_element_type=jnp.float32) + bias)
        probs_ref[0, k * block:(k + 1) * block, :] = p
        total += jnp.sum(p * jnp.log(p + 1e-9))
    acc_ref[0] += total

    @pl.when(i == nb - 1)
    def _finalize():
        ent_ref[0] = -acc_ref[0] / (nb * NSTREAMS * block)


@jax.jit
def _switch_gate(x, W, b):
    tokens, in_dim = x.shape
    num_experts = W.shape[0]
    step_rows = NSTREAMS * BLOCK
    nb = tokens // step_rows

    def _xspec(k):
        return pl.BlockSpec((BLOCK, in_dim), lambda i, k=k: (NSTREAMS * i + k, 0))

    probs, ent = pl.pallas_call(
        _gate_kernel,
        grid=(nb,),
        in_specs=[_xspec(k) for k in range(NSTREAMS)] + [
            pl.BlockSpec((num_experts, in_dim), lambda i: (0, 0)),
            pl.BlockSpec((num_experts,), lambda i: (0,)),
        ],
        out_specs=[
            pl.BlockSpec((1, step_rows, num_experts), lambda i: (i, 0, 0)),
            pl.BlockSpec(memory_space=pltpu.SMEM),
        ],
        out_shape=[
            jax.ShapeDtypeStruct((nb, step_rows, num_experts), jnp.float32),
            jax.ShapeDtypeStruct((1,), jnp.float32),
        ],
        scratch_shapes=[
            pltpu.VMEM((num_experts, in_dim), jnp.bfloat16),
            pltpu.SMEM((1,), jnp.float32),
        ],
        compiler_params=pltpu.CompilerParams(
            dimension_semantics=("arbitrary",),
        ),
    )(*([x] * NSTREAMS), W, b)
    return probs.reshape(tokens, num_experts), ent[0]


def kernel(x, W, b):
    return _switch_gate(x, W, b)


# speedup vs baseline: 4.0788x; 1.0066x over previous
import jax
import jax.numpy as jnp
from jax import lax
from jax.experimental import pallas as pl
from jax.experimental.pallas import tpu as pltpu

NSTREAMS = 2
BLOCK = 512


def _softmax_rows(logits):
    m = jnp.max(logits, axis=-1, keepdims=True)
    e = jnp.exp(logits - m)
    s = jnp.sum(e, axis=-1, keepdims=True)
    return e / s


def _gate_kernel(*refs):
    x_refs = refs[:NSTREAMS]
    w_ref, b_ref, probs_ref, w_scr = refs[NSTREAMS:]
    i = pl.program_id(0)
    block = x_refs[0].shape[0]

    @pl.when(i == 0)
    def _init():
        w_scr[...] = w_ref[...].astype(jnp.bfloat16)

    w = w_scr[...]
    bias = b_ref[...][None, :]
    for k, x_ref in enumerate(x_refs):
        p = _softmax_rows(lax.dot_general(
            x_ref[...].astype(jnp.bfloat16), w, (((1,), (1,)), ((), ())),
            preferred_element_type=jnp.float32) + bias)
        probs_ref[k * block:(k + 1) * block, :] = p


def _ent_kernel(p_ref, ent_ref, acc_ref):
    i = pl.program_id(0)
    nb = pl.num_programs(0)

    @pl.when(i == 0)
    def _init():
        acc_ref[0] = 0.0

    p = p_ref[...]
    acc_ref[0] += jnp.sum(p * jnp.log(p + 1e-9))

    @pl.when(i == nb - 1)
    def _finalize():
        ent_ref[0] = -acc_ref[0] / (nb * p_ref.shape[0])


@jax.jit
def _switch_gate(x, W, b):
    tokens, in_dim = x.shape
    num_experts = W.shape[0]
    step_rows = NSTREAMS * BLOCK
    nb = tokens // step_rows

    def _xspec(k):
        return pl.BlockSpec((BLOCK, in_dim), lambda i, k=k: (NSTREAMS * i + k, 0))

    probs = pl.pallas_call(
        _gate_kernel,
        grid=(nb,),
        in_specs=[_xspec(k) for k in range(NSTREAMS)] + [
            pl.BlockSpec((num_experts, in_dim), lambda i: (0, 0)),
            pl.BlockSpec((num_experts,), lambda i: (0,)),
        ],
        out_specs=pl.BlockSpec((step_rows, num_experts), lambda i: (i, 0)),
        out_shape=jax.ShapeDtypeStruct((tokens, num_experts), jnp.float32),
        scratch_shapes=[
            pltpu.VMEM((num_experts, in_dim), jnp.bfloat16),
        ],
        compiler_params=pltpu.CompilerParams(
            dimension_semantics=("arbitrary",),
        ),
    )(*([x] * NSTREAMS), W, b)

    ent_rows = 4096
    ent = pl.pallas_call(
        _ent_kernel,
        grid=(tokens // ent_rows,),
        in_specs=[pl.BlockSpec((ent_rows, num_experts), lambda i: (i, 0))],
        out_specs=pl.BlockSpec(memory_space=pltpu.SMEM),
        out_shape=jax.ShapeDtypeStruct((1,), jnp.float32),
        scratch_shapes=[pltpu.SMEM((1,), jnp.float32)],
        compiler_params=pltpu.CompilerParams(
            dimension_semantics=("arbitrary",),
        ),
    )(probs)
    return probs, ent[0]


def kernel(x, W, b):
    return _switch_gate(x, W, b)
